# Initial kernel scaffold; baseline (speedup 1.0000x reference)
import jax
import jax.numpy as jnp
from jax.experimental import pallas as pl


def kernel(x, edge_index, batch, W1a, b1a, W1b, b1b, W2a, b2a, W2b, b2b, W3a, b3a, W3b, b3b, Wf, bf):
    raise NotImplementedError("write your pallas kernel here")


# speedup vs baseline: 5.6613x; 5.6613x over previous
import functools

import jax
import jax.numpy as jnp
from jax import lax
from jax.experimental import pallas as pl
from jax.experimental.pallas import tpu as pltpu
from jax.experimental.pallas import tpu_sc as plsc

_N = 10000
_E = 320000
_F = 128
_H = 64
_G = 128

_NC = 2
_NS = 16
_NW = _NC * _NS
_CHUNK = 128
_EP_ROWS = 2528
_ROWS_PER_W = _EP_ROWS // _NW
_ACC_ROWS = 10240

_BLK = 1000
_NBLK = _N // _BLK


def _seg_sum_sc(y, srcm, dstm):
    mesh = plsc.VectorSubcoreMesh(core_axis_name="c", subcore_axis_name="s")

    @functools.partial(
        pl.kernel,
        mesh=mesh,
        compiler_params=pltpu.CompilerParams(use_tc_tiling_on_sc=False),
        out_type=jax.ShapeDtypeStruct((_NC, _ACC_ROWS, _H), jnp.float32),
        scratch_types=[
            pltpu.VMEM((_CHUNK,), jnp.int32),
            pltpu.VMEM((_CHUNK,), jnp.int32),
            pltpu.VMEM((_CHUNK, _H), jnp.float32),
            pltpu.VMEM_SHARED((_ACC_ROWS, _H), jnp.float32),
            pltpu.SemaphoreType.DMA,
        ],
    )
    def k(y_hbm, srcm_hbm, dstm_hbm, out_hbm, src_v, dst_v, rows_v, acc_sh, sem):
        c = lax.axis_index("c")
        s = lax.axis_index("s")
        wid = c * _NS + s

        zero16 = jnp.zeros((16,), jnp.float32)
        for r in range(_CHUNK):
            for j in range(_H // 16):
                rows_v[r, pl.ds(j * 16, 16)] = zero16
        for b in range(_ACC_ROWS // _NS // _CHUNK):
            pltpu.sync_copy(
                rows_v, acc_sh.at[pl.ds(s * (_ACC_ROWS // _NS) + b * _CHUNK, _CHUNK)]
            )
        plsc.subcore_barrier()

        def body(i, _):
            row = wid * _ROWS_PER_W + i
            pltpu.sync_copy(srcm_hbm.at[row], src_v)
            pltpu.async_copy(y_hbm.at[src_v], rows_v, sem).wait()
            pltpu.sync_copy(dstm_hbm.at[row], dst_v)
            pltpu.sync_copy(rows_v, acc_sh.at[dst_v], add=True)
            return 0

        lax.fori_loop(0, _ROWS_PER_W, body, 0)
        plsc.subcore_barrier()

        rows_out = _ACC_ROWS // _NS
        pltpu.sync_copy(
            acc_sh.at[pl.ds(s * rows_out, rows_out)],
            out_hbm.at[c, pl.ds(s * rows_out, rows_out)],
        )

    return k(y, srcm, dstm)


def _mm_body(x_ref, w_ref, o_ref):
    o_ref[...] = jnp.dot(x_ref[...], w_ref[...], preferred_element_type=jnp.float32)


def _mm_head(x, w):
    f = x.shape[1]
    return pl.pallas_call(
        _mm_body,
        grid=(_NBLK,),
        in_specs=[
            pl.BlockSpec((_BLK, f), lambda i: (i, 0)),
            pl.BlockSpec((f, _H), lambda i: (0, 0)),
        ],
        out_specs=pl.BlockSpec((_BLK, _H), lambda i: (i, 0)),
        out_shape=jax.ShapeDtypeStruct((_N, _H), jnp.float32),
    )(x, w)


def _tail_head_body(y_ref, p_ref, ba_ref, wb_ref, bb_ref, wn_ref, o_ref):
    t = jnp.maximum(y_ref[...] + p_ref[0] + p_ref[1] + ba_ref[...], 0.0)
    z = jnp.dot(t, wb_ref[...], preferred_element_type=jnp.float32) + bb_ref[...]
    o_ref[...] = jnp.dot(
        jnp.maximum(z, 0.0), wn_ref[...], preferred_element_type=jnp.float32
    )


def _tail_head(y, p, ba, wb, bb, wn):
    return pl.pallas_call(
        _tail_head_body,
        grid=(_NBLK,),
        in_specs=[
            pl.BlockSpec((_BLK, _H), lambda i: (i, 0)),
            pl.BlockSpec((_NC, _BLK, _H), lambda i: (0, i, 0)),
            pl.BlockSpec((1, _H), lambda i: (0, 0)),
            pl.BlockSpec((_H, _H), lambda i: (0, 0)),
            pl.BlockSpec((1, _H), lambda i: (0, 0)),
            pl.BlockSpec((_H, _H), lambda i: (0, 0)),
        ],
        out_specs=pl.BlockSpec((_BLK, _H), lambda i: (i, 0)),
        out_shape=jax.ShapeDtypeStruct((_N, _H), jnp.float32),
    )(y, p, ba, wb, bb, wn)


def _final_body(y_ref, p_ref, ba_ref, wb_ref, bb_ref, wf_ref, bf_ref, batch_ref,
                o_ref, acc_ref):
    i = pl.program_id(0)
    t = jnp.maximum(y_ref[...] + p_ref[0] + p_ref[1] + ba_ref[...], 0.0)
    z = jnp.dot(t, wb_ref[...], preferred_element_type=jnp.float32) + bb_ref[...]
    v = jnp.dot(z, wf_ref[...], preferred_element_type=jnp.float32)
    b2 = batch_ref[0]
    seg = lax.broadcasted_iota(jnp.int32, (_G, _BLK), 0)
    oh = (seg == b2).astype(jnp.float32)
    vv = jnp.concatenate([v, jnp.ones_like(v)], axis=1)
    contrib = jnp.dot(oh, vv, preferred_element_type=jnp.float32)

    @pl.when(i == 0)
    def _():
        acc_ref[...] = jnp.zeros_like(acc_ref)

    acc_ref[...] += contrib

    @pl.when(i == _NBLK - 1)
    def _():
        sums = acc_ref[:, 0:1]
        cnt = acc_ref[:, 1:2]
        o_ref[...] = sums / jnp.maximum(cnt, 1.0) + bf_ref[...]


def _final(y, p, ba, wb, bb, wf, bf, batch3):
    return pl.pallas_call(
        _final_body,
        grid=(_NBLK,),
        in_specs=[
            pl.BlockSpec((_BLK, _H), lambda i: (i, 0)),
            pl.BlockSpec((_NC, _BLK, _H), lambda i: (0, i, 0)),
            pl.BlockSpec((1, _H), lambda i: (0, 0)),
            pl.BlockSpec((_H, _H), lambda i: (0, 0)),
            pl.BlockSpec((1, _H), lambda i: (0, 0)),
            pl.BlockSpec((_H, 1), lambda i: (0, 0)),
            pl.BlockSpec((1, 1), lambda i: (0, 0)),
            pl.BlockSpec((1, 1, _BLK), lambda i: (i, 0, 0)),
        ],
        out_specs=pl.BlockSpec((_G, 1), lambda i: (0, 0)),
        out_shape=jax.ShapeDtypeStruct((_G, 1), jnp.float32),
        scratch_shapes=[pltpu.VMEM((_G, 2), jnp.float32)],
    )(y, p, ba, wb, bb, wf, bf, batch3)


def kernel(x, edge_index, batch, W1a, b1a, W1b, b1b, W2a, b2a, W2b, b2b,
           W3a, b3a, W3b, b3b, Wf, bf):
    pad = _EP_ROWS * _CHUNK - _E
    src = edge_index[0]
    dst = edge_index[1]
    srcm = jnp.concatenate([src, jnp.zeros((pad,), jnp.int32)]).reshape(
        _EP_ROWS, _CHUNK)
    dstm = jnp.concatenate([dst, jnp.full((pad,), _N, jnp.int32)]).reshape(
        _EP_ROWS, _CHUNK)
    batch3 = batch.reshape(_NBLK, 1, _BLK)

    y1 = _mm_head(x, W1a)
    p1 = _seg_sum_sc(y1, srcm, dstm)
    y2 = _tail_head(y1, p1, b1a.reshape(1, _H), W1b, b1b.reshape(1, _H), W2a)
    p2 = _seg_sum_sc(y2, srcm, dstm)
    y3 = _tail_head(y2, p2, b2a.reshape(1, _H), W2b, b2b.reshape(1, _H), W3a)
    p3 = _seg_sum_sc(y3, srcm, dstm)
    return _final(y3, p3, b3a.reshape(1, _H), W3b, b3b.reshape(1, _H),
                  Wf, bf.reshape(1, 1), batch3)


# speedup vs baseline: 6.0031x; 1.0604x over previous
import functools

import jax
import jax.numpy as jnp
from jax import lax
from jax.experimental import pallas as pl
from jax.experimental.pallas import tpu as pltpu
from jax.experimental.pallas import tpu_sc as plsc

_N = 10000
_E = 320000
_F = 128
_H = 64
_G = 128

_NC = 2
_NS = 16
_NW = _NC * _NS
_CHUNK = 128
_ROWS_PER_W = 80
_EP_ROWS = _NW * _ROWS_PER_W
_ACC_ROWS = 10240
_NBUF = 8
_NIF = 4

_BLK = 1000
_NBLK = _N // _BLK


def _seg_sum_sc(y, srcm, dstm):
    mesh = plsc.VectorSubcoreMesh(core_axis_name="c", subcore_axis_name="s")

    @functools.partial(
        pl.kernel,
        mesh=mesh,
        compiler_params=pltpu.CompilerParams(use_tc_tiling_on_sc=False),
        out_type=jax.ShapeDtypeStruct((_NC, _ACC_ROWS, _H), jnp.float32),
        scratch_types=[
            pltpu.VMEM((_ROWS_PER_W, _CHUNK), jnp.int32),
            pltpu.VMEM((_ROWS_PER_W, _CHUNK), jnp.int32),
            pltpu.VMEM((_NBUF, _CHUNK, _H), jnp.float32),
            pltpu.VMEM_SHARED((_ACC_ROWS, _H), jnp.float32),
            pltpu.SemaphoreType.DMA((_NBUF,)),
            pltpu.SemaphoreType.DMA((_NBUF,)),
        ],
    )
    def k(y_hbm, srcm_hbm, dstm_hbm, out_hbm, sidx_v, didx_v, rows_v, acc_sh,
          gsem, ssem):
        c = lax.axis_index("c")
        s = lax.axis_index("s")
        wid = c * _NS + s

        pltpu.sync_copy(srcm_hbm.at[pl.ds(wid * _ROWS_PER_W, _ROWS_PER_W)], sidx_v)
        pltpu.sync_copy(dstm_hbm.at[pl.ds(wid * _ROWS_PER_W, _ROWS_PER_W)], didx_v)

        zero16 = jnp.zeros((16,), jnp.float32)
        for r in range(_CHUNK):
            for j in range(_H // 16):
                rows_v[0, r, pl.ds(j * 16, 16)] = zero16
        for b in range(_ACC_ROWS // _NS // _CHUNK):
            pltpu.sync_copy(
                rows_v.at[0],
                acc_sh.at[pl.ds(s * (_ACC_ROWS // _NS) + b * _CHUNK, _CHUNK)],
            )
        plsc.subcore_barrier()

        def gather(ch):
            b = ch % _NBUF
            pltpu.async_copy(y_hbm.at[sidx_v.at[ch]], rows_v.at[b], gsem.at[b])

        def gather_wait(ch):
            b = ch % _NBUF
            pltpu.make_async_copy(
                y_hbm.at[sidx_v.at[ch]], rows_v.at[b], gsem.at[b]
            ).wait()

        def scatter(ch):
            b = ch % _NBUF
            pltpu.async_copy(
                rows_v.at[b], acc_sh.at[didx_v.at[ch]], ssem.at[b], add=True
            )

        def scatter_wait(ch):
            b = ch % _NBUF
            pltpu.make_async_copy(
                rows_v.at[b], acc_sh.at[didx_v.at[ch]], ssem.at[b]
            ).wait()

        for ch in range(_NIF):
            gather(ch)
        for ch in range(_ROWS_PER_W):
            nxt = ch + _NIF
            if nxt < _ROWS_PER_W:
                if nxt >= _NBUF:
                    scatter_wait(nxt - _NBUF)
                gather(nxt)
            gather_wait(ch)
            scatter(ch)
        for ch in range(_ROWS_PER_W - _NBUF, _ROWS_PER_W):
            scatter_wait(ch)
        plsc.subcore_barrier()

        rows_out = _ACC_ROWS // _NS
        pltpu.sync_copy(
            acc_sh.at[pl.ds(s * rows_out, rows_out)],
            out_hbm.at[c, pl.ds(s * rows_out, rows_out)],
        )

    return k(y, srcm, dstm)


def _mm_body(x_ref, w_ref, o_ref):
    o_ref[...] = jnp.dot(x_ref[...], w_ref[...], preferred_element_type=jnp.float32)


def _mm_head(x, w):
    f = x.shape[1]
    return pl.pallas_call(
        _mm_body,
        grid=(_NBLK,),
        in_specs=[
            pl.BlockSpec((_BLK, f), lambda i: (i, 0)),
            pl.BlockSpec((f, _H), lambda i: (0, 0)),
        ],
        out_specs=pl.BlockSpec((_BLK, _H), lambda i: (i, 0)),
        out_shape=jax.ShapeDtypeStruct((_N, _H), jnp.float32),
    )(x, w)


def _tail_head_body(y_ref, p_ref, ba_ref, wb_ref, bb_ref, wn_ref, o_ref):
    t = jnp.maximum(y_ref[...] + p_ref[0] + p_ref[1] + ba_ref[...], 0.0)
    z = jnp.dot(t, wb_ref[...], preferred_element_type=jnp.float32) + bb_ref[...]
    o_ref[...] = jnp.dot(
        jnp.maximum(z, 0.0), wn_ref[...], preferred_element_type=jnp.float32
    )


def _tail_head(y, p, ba, wb, bb, wn):
    return pl.pallas_call(
        _tail_head_body,
        grid=(_NBLK,),
        in_specs=[
            pl.BlockSpec((_BLK, _H), lambda i: (i, 0)),
            pl.BlockSpec((_NC, _BLK, _H), lambda i: (0, i, 0)),
            pl.BlockSpec((1, _H), lambda i: (0, 0)),
            pl.BlockSpec((_H, _H), lambda i: (0, 0)),
            pl.BlockSpec((1, _H), lambda i: (0, 0)),
            pl.BlockSpec((_H, _H), lambda i: (0, 0)),
        ],
        out_specs=pl.BlockSpec((_BLK, _H), lambda i: (i, 0)),
        out_shape=jax.ShapeDtypeStruct((_N, _H), jnp.float32),
    )(y, p, ba, wb, bb, wn)


def _final_body(y_ref, p_ref, ba_ref, wb_ref, bb_ref, wf_ref, bf_ref, batch_ref,
                o_ref, acc_ref):
    i = pl.program_id(0)
    t = jnp.maximum(y_ref[...] + p_ref[0] + p_ref[1] + ba_ref[...], 0.0)
    z = jnp.dot(t, wb_ref[...], preferred_element_type=jnp.float32) + bb_ref[...]
    v = jnp.dot(z, wf_ref[...], preferred_element_type=jnp.float32)
    b2 = batch_ref[0]
    seg = lax.broadcasted_iota(jnp.int32, (_G, _BLK), 0)
    oh = (seg == b2).astype(jnp.float32)
    vv = jnp.concatenate([v, jnp.ones_like(v)], axis=1)
    contrib = jnp.dot(oh, vv, preferred_element_type=jnp.float32)

    @pl.when(i == 0)
    def _():
        acc_ref[...] = jnp.zeros_like(acc_ref)

    acc_ref[...] += contrib

    @pl.when(i == _NBLK - 1)
    def _():
        sums = acc_ref[:, 0:1]
        cnt = acc_ref[:, 1:2]
        o_ref[...] = sums / jnp.maximum(cnt, 1.0) + bf_ref[...]


def _final(y, p, ba, wb, bb, wf, bf, batch3):
    return pl.pallas_call(
        _final_body,
        grid=(_NBLK,),
        in_specs=[
            pl.BlockSpec((_BLK, _H), lambda i: (i, 0)),
            pl.BlockSpec((_NC, _BLK, _H), lambda i: (0, i, 0)),
            pl.BlockSpec((1, _H), lambda i: (0, 0)),
            pl.BlockSpec((_H, _H), lambda i: (0, 0)),
            pl.BlockSpec((1, _H), lambda i: (0, 0)),
            pl.BlockSpec((_H, 1), lambda i: (0, 0)),
            pl.BlockSpec((1, 1), lambda i: (0, 0)),
            pl.BlockSpec((1, 1, _BLK), lambda i: (i, 0, 0)),
        ],
        out_specs=pl.BlockSpec((_G, 1), lambda i: (0, 0)),
        out_shape=jax.ShapeDtypeStruct((_G, 1), jnp.float32),
        scratch_shapes=[pltpu.VMEM((_G, 2), jnp.float32)],
    )(y, p, ba, wb, bb, wf, bf, batch3)


def kernel(x, edge_index, batch, W1a, b1a, W1b, b1b, W2a, b2a, W2b, b2b,
           W3a, b3a, W3b, b3b, Wf, bf):
    pad = _EP_ROWS * _CHUNK - _E
    src = edge_index[0]
    dst = edge_index[1]
    srcm = jnp.concatenate([src, jnp.zeros((pad,), jnp.int32)]).reshape(
        _EP_ROWS, _CHUNK)
    dstm = jnp.concatenate([dst, jnp.full((pad,), _N, jnp.int32)]).reshape(
        _EP_ROWS, _CHUNK)
    batch3 = batch.reshape(_NBLK, 1, _BLK)

    y1 = _mm_head(x, W1a)
    p1 = _seg_sum_sc(y1, srcm, dstm)
    y2 = _tail_head(y1, p1, b1a.reshape(1, _H), W1b, b1b.reshape(1, _H), W2a)
    p2 = _seg_sum_sc(y2, srcm, dstm)
    y3 = _tail_head(y2, p2, b2a.reshape(1, _H), W2b, b2b.reshape(1, _H), W3a)
    p3 = _seg_sum_sc(y3, srcm, dstm)
    return _final(y3, p3, b3a.reshape(1, _H), W3b, b3b.reshape(1, _H),
                  Wf, bf.reshape(1, 1), batch3)


# speedup vs baseline: 17.0194x; 2.8351x over previous
import functools

import jax
import jax.numpy as jnp
from jax import lax
from jax.experimental import pallas as pl
from jax.experimental.pallas import tpu as pltpu
from jax.experimental.pallas import tpu_sc as plsc

_N = 10000
_E = 320000
_F = 128
_H = 64
_G = 128

_NC = 2
_NS = 16
_NW = _NC * _NS
_CHUNK = 128
_ROWS_PER_W = 80
_EP_ROWS = _NW * _ROWS_PER_W
_ACC_ROWS = 10240
_NBUF = 8
_NIF = 4

_BLK = 1000
_NBLK = _N // _BLK


def _seg_sum_sc(y, srcm, dstm):
    mesh = plsc.VectorSubcoreMesh(core_axis_name="c", subcore_axis_name="s")

    @functools.partial(
        pl.kernel,
        mesh=mesh,
        compiler_params=pltpu.CompilerParams(use_tc_tiling_on_sc=False),
        out_type=jax.ShapeDtypeStruct((_NC, _ACC_ROWS, _H), jnp.float32),
        scratch_types=[
            pltpu.VMEM((_ROWS_PER_W, _CHUNK), jnp.int32),
            pltpu.VMEM((_ROWS_PER_W, _CHUNK), jnp.int32),
            pltpu.VMEM((_NBUF, _CHUNK, _H), jnp.float32),
            pltpu.VMEM_SHARED((_ACC_ROWS, _H), jnp.float32),
            pltpu.SemaphoreType.DMA((_NBUF,)),
            pltpu.SemaphoreType.DMA((_NBUF,)),
        ],
    )
    def k(y_hbm, srcm_hbm, dstm_hbm, out_hbm, sidx_v, didx_v, rows_v, acc_sh,
          gsem, ssem):
        c = lax.axis_index("c")
        s = lax.axis_index("s")
        wid = c * _NS + s

        pltpu.sync_copy(srcm_hbm.at[pl.ds(wid * _ROWS_PER_W, _ROWS_PER_W)], sidx_v)
        pltpu.sync_copy(dstm_hbm.at[pl.ds(wid * _ROWS_PER_W, _ROWS_PER_W)], didx_v)

        zero16 = jnp.zeros((16,), jnp.float32)
        for r in range(_CHUNK):
            for j in range(_H // 16):
                rows_v[0, r, pl.ds(j * 16, 16)] = zero16
        for b in range(_ACC_ROWS // _NS // _CHUNK):
            pltpu.sync_copy(
                rows_v.at[0],
                acc_sh.at[pl.ds(s * (_ACC_ROWS // _NS) + b * _CHUNK, _CHUNK)],
            )
        plsc.subcore_barrier()

        def gather(ch):
            b = ch % _NBUF
            pltpu.async_copy(y_hbm.at[sidx_v.at[ch]], rows_v.at[b], gsem.at[b])

        def gather_wait(ch):
            b = ch % _NBUF
            pltpu.make_async_copy(
                y_hbm.at[sidx_v.at[ch]], rows_v.at[b], gsem.at[b]
            ).wait()

        def scatter(ch):
            b = ch % _NBUF
            pltpu.async_copy(
                rows_v.at[b], acc_sh.at[didx_v.at[ch]], ssem.at[b], add=True
            )

        def scatter_wait(ch):
            b = ch % _NBUF
            pltpu.make_async_copy(
                rows_v.at[b], acc_sh.at[didx_v.at[ch]], ssem.at[b]
            ).wait()

        for ch in range(_NIF):
            gather(ch)
        for ch in range(_ROWS_PER_W):
            nxt = ch + _NIF
            if nxt < _ROWS_PER_W:
                if nxt >= _NBUF:
                    scatter_wait(nxt - _NBUF)
                gather(nxt)
            gather_wait(ch)
            scatter(ch)
        for ch in range(_ROWS_PER_W - _NBUF, _ROWS_PER_W):
            scatter_wait(ch)
        plsc.subcore_barrier()

        rows_out = _ACC_ROWS // _NS
        pltpu.sync_copy(
            acc_sh.at[pl.ds(s * rows_out, rows_out)],
            out_hbm.at[c, pl.ds(s * rows_out, rows_out)],
        )

    return k(y, srcm, dstm)


def _mm_body(x_ref, w_ref, o_ref):
    o_ref[...] = jnp.dot(x_ref[...], w_ref[...], preferred_element_type=jnp.float32)


def _mm_head(x, w):
    f = x.shape[1]
    return pl.pallas_call(
        _mm_body,
        grid=(_NBLK,),
        in_specs=[
            pl.BlockSpec((_BLK, f), lambda i: (i, 0)),
            pl.BlockSpec((f, _H), lambda i: (0, 0)),
        ],
        out_specs=pl.BlockSpec((_BLK, _H), lambda i: (i, 0)),
        out_shape=jax.ShapeDtypeStruct((_N, _H), jnp.float32),
    )(x, w)


def _tail_head_body(y_ref, p_ref, ba_ref, wb_ref, bb_ref, wn_ref, o_ref):
    t = jnp.maximum(y_ref[...] + p_ref[0] + p_ref[1] + ba_ref[...], 0.0)
    z = jnp.dot(t, wb_ref[...], preferred_element_type=jnp.float32) + bb_ref[...]
    o_ref[...] = jnp.dot(
        jnp.maximum(z, 0.0), wn_ref[...], preferred_element_type=jnp.float32
    )


def _tail_head(y, p, ba, wb, bb, wn):
    return pl.pallas_call(
        _tail_head_body,
        grid=(_NBLK,),
        in_specs=[
            pl.BlockSpec((_BLK, _H), lambda i: (i, 0)),
            pl.BlockSpec((_NC, _BLK, _H), lambda i: (0, i, 0)),
            pl.BlockSpec((1, _H), lambda i: (0, 0)),
            pl.BlockSpec((_H, _H), lambda i: (0, 0)),
            pl.BlockSpec((1, _H), lambda i: (0, 0)),
            pl.BlockSpec((_H, _H), lambda i: (0, 0)),
        ],
        out_specs=pl.BlockSpec((_BLK, _H), lambda i: (i, 0)),
        out_shape=jax.ShapeDtypeStruct((_N, _H), jnp.float32),
    )(y, p, ba, wb, bb, wn)


def _final_body(y_ref, p_ref, ba_ref, wb_ref, bb_ref, wf_ref, bf_ref, batch_ref,
                o_ref, acc_ref):
    i = pl.program_id(0)
    t = jnp.maximum(y_ref[...] + p_ref[0] + p_ref[1] + ba_ref[...], 0.0)
    z = jnp.dot(t, wb_ref[...], preferred_element_type=jnp.float32) + bb_ref[...]
    v = jnp.dot(z, wf_ref[...], preferred_element_type=jnp.float32)
    b2 = batch_ref[0]
    seg = lax.broadcasted_iota(jnp.int32, (_G, _BLK), 0)
    oh = (seg == b2).astype(jnp.float32)
    vv = jnp.concatenate([v, jnp.ones_like(v)], axis=1)
    contrib = jnp.dot(oh, vv, preferred_element_type=jnp.float32)

    @pl.when(i == 0)
    def _():
        acc_ref[...] = jnp.zeros_like(acc_ref)

    acc_ref[...] += contrib

    @pl.when(i == _NBLK - 1)
    def _():
        sums = acc_ref[:, 0:1]
        cnt = acc_ref[:, 1:2]
        o_ref[...] = sums / jnp.maximum(cnt, 1.0) + bf_ref[...]


def _final(y, p, ba, wb, bb, wf, bf, batch3):
    return pl.pallas_call(
        _final_body,
        grid=(_NBLK,),
        in_specs=[
            pl.BlockSpec((_BLK, _H), lambda i: (i, 0)),
            pl.BlockSpec((_NC, _BLK, _H), lambda i: (0, i, 0)),
            pl.BlockSpec((1, _H), lambda i: (0, 0)),
            pl.BlockSpec((_H, _H), lambda i: (0, 0)),
            pl.BlockSpec((1, _H), lambda i: (0, 0)),
            pl.BlockSpec((_H, 1), lambda i: (0, 0)),
            pl.BlockSpec((1, 1), lambda i: (0, 0)),
            pl.BlockSpec((1, 1, _BLK), lambda i: (i, 0, 0)),
        ],
        out_specs=pl.BlockSpec((_G, 1), lambda i: (0, 0)),
        out_shape=jax.ShapeDtypeStruct((_G, 1), jnp.float32),
        scratch_shapes=[pltpu.VMEM((_G, 2), jnp.float32)],
    )(y, p, ba, wb, bb, wf, bf, batch3)


def kernel(x, edge_index, batch, W1a, b1a, W1b, b1b, W2a, b2a, W2b, b2b,
           W3a, b3a, W3b, b3b, Wf, bf):
    pad = _EP_ROWS * _CHUNK - _E
    src = edge_index[0]
    dst = edge_index[1]
    pad_ids = lax.iota(jnp.int32, pad)
    srcm = jnp.concatenate([src, pad_ids % _N]).reshape(_EP_ROWS, _CHUNK)
    dstm = jnp.concatenate([dst, _N + pad_ids % (_ACC_ROWS - _N)]).reshape(
        _EP_ROWS, _CHUNK)
    batch3 = batch.reshape(_NBLK, 1, _BLK)

    y1 = _mm_head(x, W1a)
    p1 = _seg_sum_sc(y1, srcm, dstm)
    y2 = _tail_head(y1, p1, b1a.reshape(1, _H), W1b, b1b.reshape(1, _H), W2a)
    p2 = _seg_sum_sc(y2, srcm, dstm)
    y3 = _tail_head(y2, p2, b2a.reshape(1, _H), W2b, b2b.reshape(1, _H), W3a)
    p3 = _seg_sum_sc(y3, srcm, dstm)
    return _final(y3, p3, b3a.reshape(1, _H), W3b, b3b.reshape(1, _H),
                  Wf, bf.reshape(1, 1), batch3)
